# baseline (speedup 1.0000x reference)
import functools

import jax
import jax.numpy as jnp
from jax import lax
from jax.experimental import pallas as pl
from jax.experimental.pallas import tpu as pltpu
from jax.experimental.pallas import tpu_sc as plsc

_NC = 2
_NS = 16
_NW = _NC * _NS
_L = 16

_C = 8

_N0 = 48
_N1 = 32


def _build_kernel(B_pad, S, D, scale, n0, n1):
    C = _C
    total_chunks = B_pad // C
    assert _NS * (n0 + n1) == total_chunks
    n_max = max(n0, n1)
    mesh = plsc.VectorSubcoreMesh(core_axis_name="c", subcore_axis_name="s")

    @functools.partial(
        pl.kernel,
        out_type=jax.ShapeDtypeStruct((B_pad, D), jnp.float32),
        mesh=mesh,
        scratch_types=[
            pltpu.VMEM((n_max, C * S), jnp.int32),
            pltpu.VMEM((C * S, D), jnp.float32),
            pltpu.VMEM((C * S, D), jnp.float32),
            pltpu.VMEM((C, D), jnp.float32),
            pltpu.VMEM((C, D), jnp.float32),
            pltpu.SemaphoreType.DMA,
            pltpu.SemaphoreType.DMA,
            pltpu.SemaphoreType.DMA,
            pltpu.SemaphoreType.DMA,
        ],
    )
    def aggr(feat_hbm, nidx_hbm, out_hbm, idx_ref, g0, g1, o0, o1,
             sg0, sg1, so0, so1):
        cid = lax.axis_index("c")
        sid = lax.axis_index("s")
        bufs = ((g0, sg0, o0, so0), (g1, sg1, o1, so1))

        def run(n_chunks, start_chunk):
            pltpu.sync_copy(nidx_hbm.at[pl.ds(start_chunk, n_chunks)],
                            idx_ref.at[pl.ds(0, n_chunks)])
            base_row = start_chunk * C

            pltpu.async_copy(feat_hbm.at[idx_ref.at[0]], g0, sg0)
            pltpu.async_copy(feat_hbm.at[idx_ref.at[1]], g1, sg1)

            def pair_body(p, carry):
                j = p * 2
                for b, (g, sg, o, so) in enumerate(bufs):
                    jj = j + b
                    pltpu.make_async_copy(
                        feat_hbm.at[idx_ref.at[jj]], g, sg).wait()

                    @pl.when(p > 0)
                    def _wait_out():
                        pltpu.make_async_copy(
                            o, out_hbm.at[pl.ds(base_row + (jj - 2) * C, C)],
                            so).wait()

                    def reduce_row(r, c2):
                        row = r * S
                        for v in range(D // _L):
                            sl = pl.ds(v * _L, _L)
                            acc = g[row, sl]
                            for s in range(1, S):
                                acc = acc + g[row + s, sl]
                            o[r, sl] = acc * scale
                        return c2

                    lax.fori_loop(0, C, reduce_row, 0, unroll=False)
                    pltpu.async_copy(
                        o, out_hbm.at[pl.ds(base_row + jj * C, C)], so)

                    @pl.when(jj + 2 < n_chunks)
                    def _next_gather():
                        pltpu.async_copy(
                            feat_hbm.at[idx_ref.at[jj + 2]], g, sg)

                return carry

            lax.fori_loop(0, n_chunks // 2, pair_body, 0, unroll=False)
            pltpu.make_async_copy(
                o0, out_hbm.at[pl.ds(base_row + (n_chunks - 2) * C, C)], so0
            ).wait()
            pltpu.make_async_copy(
                o1, out_hbm.at[pl.ds(base_row + (n_chunks - 1) * C, C)], so1
            ).wait()

        @pl.when(cid == 0)
        def _core0():
            run(n0, sid * n0)

        @pl.when(cid == 1)
        def _core1():
            run(n1, _NS * n0 + sid * n1)

    return aggr


def kernel(features, nodes, neigh_idx, num_sample):
    N, D = features.shape
    B, S = neigh_idx.shape
    B_pad = _C * _NS * (_N0 + _N1)
    assert B_pad >= B
    nidx = neigh_idx.astype(jnp.int32)
    if B_pad != B:
        nidx = jnp.pad(nidx, ((0, B_pad - B), (0, 0)))
    nidx = nidx.reshape(B_pad // _C, _C * S)

    aggr = _build_kernel(B_pad, S, D, 1.0 / float(S), _N0, _N1)
    out = aggr(features, nidx)
    return out[:B]


# speedup vs baseline: 1.0601x; 1.0601x over previous
import functools

import jax
import jax.numpy as jnp
from jax import lax
from jax.experimental import pallas as pl
from jax.experimental.pallas import tpu as pltpu
from jax.experimental.pallas import tpu_sc as plsc

_NC = 2
_NS = 16
_NW = _NC * _NS
_L = 16

_C = 8

_N0 = 64
_N1 = 16


def _build_kernel(B_pad, S, D, scale, n0, n1):
    C = _C
    total_chunks = B_pad // C
    assert _NS * (n0 + n1) == total_chunks
    n_max = max(n0, n1)
    mesh = plsc.VectorSubcoreMesh(core_axis_name="c", subcore_axis_name="s")

    @functools.partial(
        pl.kernel,
        out_type=jax.ShapeDtypeStruct((B_pad, D), jnp.float32),
        mesh=mesh,
        scratch_types=[
            pltpu.VMEM((n_max, C * S), jnp.int32),
            pltpu.VMEM((C * S, D), jnp.float32),
            pltpu.VMEM((C * S, D), jnp.float32),
            pltpu.VMEM((C, D), jnp.float32),
            pltpu.VMEM((C, D), jnp.float32),
            pltpu.SemaphoreType.DMA,
            pltpu.SemaphoreType.DMA,
            pltpu.SemaphoreType.DMA,
            pltpu.SemaphoreType.DMA,
        ],
    )
    def aggr(feat_hbm, nidx_hbm, out_hbm, idx_ref, g0, g1, o0, o1,
             sg0, sg1, so0, so1):
        cid = lax.axis_index("c")
        sid = lax.axis_index("s")
        bufs = ((g0, sg0, o0, so0), (g1, sg1, o1, so1))

        def run(n_chunks, start_chunk):
            pltpu.sync_copy(nidx_hbm.at[pl.ds(start_chunk, n_chunks)],
                            idx_ref.at[pl.ds(0, n_chunks)])
            base_row = start_chunk * C

            pltpu.async_copy(feat_hbm.at[idx_ref.at[0]], g0, sg0)
            pltpu.async_copy(feat_hbm.at[idx_ref.at[1]], g1, sg1)

            def pair_body(p, carry):
                j = p * 2
                for b, (g, sg, o, so) in enumerate(bufs):
                    jj = j + b
                    pltpu.make_async_copy(
                        feat_hbm.at[idx_ref.at[jj]], g, sg).wait()

                    @pl.when(p > 0)
                    def _wait_out():
                        pltpu.make_async_copy(
                            o, out_hbm.at[pl.ds(base_row + (jj - 2) * C, C)],
                            so).wait()

                    def reduce_row(r, c2):
                        row = r * S
                        for v in range(D // _L):
                            sl = pl.ds(v * _L, _L)
                            acc = g[row, sl]
                            for s in range(1, S):
                                acc = acc + g[row + s, sl]
                            o[r, sl] = acc * scale
                        return c2

                    lax.fori_loop(0, C, reduce_row, 0, unroll=False)
                    pltpu.async_copy(
                        o, out_hbm.at[pl.ds(base_row + jj * C, C)], so)

                    @pl.when(jj + 2 < n_chunks)
                    def _next_gather():
                        pltpu.async_copy(
                            feat_hbm.at[idx_ref.at[jj + 2]], g, sg)

                return carry

            lax.fori_loop(0, n_chunks // 2, pair_body, 0, unroll=False)
            pltpu.make_async_copy(
                o0, out_hbm.at[pl.ds(base_row + (n_chunks - 2) * C, C)], so0
            ).wait()
            pltpu.make_async_copy(
                o1, out_hbm.at[pl.ds(base_row + (n_chunks - 1) * C, C)], so1
            ).wait()

        @pl.when(cid == 0)
        def _core0():
            run(n0, sid * n0)

        @pl.when(cid == 1)
        def _core1():
            run(n1, _NS * n0 + sid * n1)

    return aggr


def kernel(features, nodes, neigh_idx, num_sample):
    N, D = features.shape
    B, S = neigh_idx.shape
    B_pad = _C * _NS * (_N0 + _N1)
    assert B_pad >= B
    nidx = neigh_idx.astype(jnp.int32)
    if B_pad != B:
        nidx = jnp.pad(nidx, ((0, B_pad - B), (0, 0)))
    nidx = nidx.reshape(B_pad // _C, _C * S)

    aggr = _build_kernel(B_pad, S, D, 1.0 / float(S), _N0, _N1)
    out = aggr(features, nidx)
    return out[:B]


# speedup vs baseline: 1.0758x; 1.0149x over previous
import functools

import jax
import jax.numpy as jnp
from jax import lax
from jax.experimental import pallas as pl
from jax.experimental.pallas import tpu as pltpu
from jax.experimental.pallas import tpu_sc as plsc

_NC = 2
_NS = 16
_NW = _NC * _NS
_L = 16

_C = 8

_N0 = 72
_N1 = 8


def _build_kernel(B_pad, S, D, scale, n0, n1):
    C = _C
    total_chunks = B_pad // C
    assert _NS * (n0 + n1) == total_chunks
    n_max = max(n0, n1)
    mesh = plsc.VectorSubcoreMesh(core_axis_name="c", subcore_axis_name="s")

    @functools.partial(
        pl.kernel,
        out_type=jax.ShapeDtypeStruct((B_pad, D), jnp.float32),
        mesh=mesh,
        scratch_types=[
            pltpu.VMEM((n_max, C * S), jnp.int32),
            pltpu.VMEM((C * S, D), jnp.float32),
            pltpu.VMEM((C * S, D), jnp.float32),
            pltpu.VMEM((C, D), jnp.float32),
            pltpu.VMEM((C, D), jnp.float32),
            pltpu.SemaphoreType.DMA,
            pltpu.SemaphoreType.DMA,
            pltpu.SemaphoreType.DMA,
            pltpu.SemaphoreType.DMA,
        ],
    )
    def aggr(feat_hbm, nidx_hbm, out_hbm, idx_ref, g0, g1, o0, o1,
             sg0, sg1, so0, so1):
        cid = lax.axis_index("c")
        sid = lax.axis_index("s")
        bufs = ((g0, sg0, o0, so0), (g1, sg1, o1, so1))

        def run(n_chunks, start_chunk):
            pltpu.sync_copy(nidx_hbm.at[pl.ds(start_chunk, n_chunks)],
                            idx_ref.at[pl.ds(0, n_chunks)])
            base_row = start_chunk * C

            pltpu.async_copy(feat_hbm.at[idx_ref.at[0]], g0, sg0)
            pltpu.async_copy(feat_hbm.at[idx_ref.at[1]], g1, sg1)

            def pair_body(p, carry):
                j = p * 2
                for b, (g, sg, o, so) in enumerate(bufs):
                    jj = j + b
                    pltpu.make_async_copy(
                        feat_hbm.at[idx_ref.at[jj]], g, sg).wait()

                    @pl.when(p > 0)
                    def _wait_out():
                        pltpu.make_async_copy(
                            o, out_hbm.at[pl.ds(base_row + (jj - 2) * C, C)],
                            so).wait()

                    def reduce_row(r, c2):
                        row = r * S
                        for v in range(D // _L):
                            sl = pl.ds(v * _L, _L)
                            acc = g[row, sl]
                            for s in range(1, S):
                                acc = acc + g[row + s, sl]
                            o[r, sl] = acc * scale
                        return c2

                    lax.fori_loop(0, C, reduce_row, 0, unroll=False)
                    pltpu.async_copy(
                        o, out_hbm.at[pl.ds(base_row + jj * C, C)], so)

                    @pl.when(jj + 2 < n_chunks)
                    def _next_gather():
                        pltpu.async_copy(
                            feat_hbm.at[idx_ref.at[jj + 2]], g, sg)

                return carry

            lax.fori_loop(0, n_chunks // 2, pair_body, 0, unroll=False)
            pltpu.make_async_copy(
                o0, out_hbm.at[pl.ds(base_row + (n_chunks - 2) * C, C)], so0
            ).wait()
            pltpu.make_async_copy(
                o1, out_hbm.at[pl.ds(base_row + (n_chunks - 1) * C, C)], so1
            ).wait()

        @pl.when(cid == 0)
        def _core0():
            run(n0, sid * n0)

        @pl.when(cid == 1)
        def _core1():
            run(n1, _NS * n0 + sid * n1)

    return aggr


def kernel(features, nodes, neigh_idx, num_sample):
    N, D = features.shape
    B, S = neigh_idx.shape
    B_pad = _C * _NS * (_N0 + _N1)
    assert B_pad >= B
    nidx = neigh_idx.astype(jnp.int32)
    if B_pad != B:
        nidx = jnp.pad(nidx, ((0, B_pad - B), (0, 0)))
    nidx = nidx.reshape(B_pad // _C, _C * S)

    aggr = _build_kernel(B_pad, S, D, 1.0 / float(S), _N0, _N1)
    out = aggr(features, nidx)
    return out[:B]
